# baseline (speedup 1.0000x reference)
import jax
import jax.numpy as jnp
from jax.experimental import pallas as pl

S_BLK = 1024


def _add_kernel(x_ref, pos_ref, o_ref):
    s = pl.program_id(0)
    o_ref[...] = x_ref[...] + pos_ref[pl.ds(s * S_BLK, S_BLK), :][None]


def kernel(x, pos_table):
    batch, seq_len, d_model = x.shape
    n_s = seq_len // S_BLK
    return pl.pallas_call(
        _add_kernel,
        grid=(n_s, batch),
        in_specs=[
            pl.BlockSpec((1, S_BLK, d_model), lambda s, b: (b, s, 0)),
            pl.BlockSpec((seq_len, d_model), lambda s, b: (0, 0)),
        ],
        out_specs=pl.BlockSpec((1, S_BLK, d_model), lambda s, b: (b, s, 0)),
        out_shape=jax.ShapeDtypeStruct((batch, seq_len, d_model), x.dtype),
    )(x, pos_table)
